# baseline (speedup 1.0000x reference)
import jax
import jax.numpy as jnp
from jax.experimental import pallas as pl
from jax.experimental.pallas import tpu as pltpu

EMBED = 256
HEADS = 8
HD = EMBED // HEADS
MLP = 1024
LAYERS = 3
EPS = 1e-5
BB = 4
TQ = 32


def _ln(x, g, b):
    m = jnp.mean(x, axis=-1, keepdims=True)
    v = jnp.mean((x - m) ** 2, axis=-1, keepdims=True)
    return (x - m) * jax.lax.rsqrt(v + EPS) * g + b


def _attn(h, kvs, WqT, bq, Wk, WvT, bv, WoT, bo):
    q = h @ WqT + bq
    o_parts = []
    for bidx in range(BB):
        qb = q[TQ * bidx:TQ * (bidx + 1)]
        kv = kvs[bidx]
        A = jnp.concatenate(
            [qb[:, HD * i:HD * (i + 1)] @ Wk[HD * i:HD * (i + 1), :]
             for i in range(HEADS)], axis=0)
        s = jax.lax.dot_general(A.astype(kv.dtype), kv, (((1,), (1,)), ((), ())),
                                preferred_element_type=jnp.float32)
        e = jnp.exp2(s)
        rs = jnp.sum(e, axis=-1, keepdims=True)
        U = jnp.dot(e.astype(kv.dtype), kv,
                    preferred_element_type=jnp.float32)
        U = U * (1.0 / rs)
        o_parts.append(jnp.concatenate(
            [U[HD * i:HD * (i + 1), :] @ WvT[:, HD * i:HD * (i + 1)]
             for i in range(HEADS)], axis=1))
    return (jnp.concatenate(o_parts, axis=0) + bv) @ WoT + bo


def _decoder_kernel(vlm_ref, img_ref,
                    ln1g, ln1b, saWqT, sabq, saWk, saWvT, sabv, saWoT, sabo,
                    ln2g, ln2b, caWqT, cabq, caWk, caWvT, cabv, caWoT, cabo,
                    ln3g, ln3b, W1T, b1, W2T, b2,
                    o_ref):
    x = vlm_ref[:].reshape(BB * TQ, EMBED)
    imgs = [img_ref[i] for i in range(BB)]
    for l in range(LAYERS):
        h = _ln(x, ln1g[l], ln1b[l])
        hs = [h[TQ * i:TQ * (i + 1)] for i in range(BB)]
        x = x + _attn(h, hs, saWqT[l], sabq[l], saWk[l], saWvT[l], sabv[l],
                      saWoT[l], sabo[l])
        h = _ln(x, ln2g[l], ln2b[l])
        x = x + _attn(h, imgs, caWqT[l], cabq[l], caWk[l], caWvT[l], cabv[l],
                      caWoT[l], cabo[l])
        h = _ln(x, ln3g[l], ln3b[l])
        h = jax.nn.gelu(h @ W1T[l] + b1[l], approximate=True)
        x = x + h @ W2T[l] + b2[l]
    o_ref[:] = x.reshape(BB, TQ, EMBED)


def _stack(ls, f):
    return jnp.stack([f(lp) for lp in ls])


@jax.jit
def kernel(vlm_features, image_embeddings, params):
    B, tq, D = vlm_features.shape
    TK = image_embeddings.shape[1]
    ls = params["layers"]
    scale = jnp.float32(1.4426950408889634) / jnp.sqrt(jnp.float32(HD))

    ws = [
        _stack(ls, lambda p: p["ln1_g"].reshape(1, D)),
        _stack(ls, lambda p: p["ln1_b"].reshape(1, D)),
        _stack(ls, lambda p: p["sa"]["Wq"].T * scale),
        _stack(ls, lambda p: (p["sa"]["bq"] * scale).reshape(1, D)),
        _stack(ls, lambda p: p["sa"]["Wk"]),
        _stack(ls, lambda p: p["sa"]["Wv"].T),
        _stack(ls, lambda p: p["sa"]["bv"].reshape(1, D)),
        _stack(ls, lambda p: p["sa"]["Wo"].T),
        _stack(ls, lambda p: p["sa"]["bo"].reshape(1, D)),
        _stack(ls, lambda p: p["ln2_g"].reshape(1, D)),
        _stack(ls, lambda p: p["ln2_b"].reshape(1, D)),
        _stack(ls, lambda p: p["ca"]["Wq"].T * scale),
        _stack(ls, lambda p: (p["ca"]["bq"] * scale).reshape(1, D)),
        _stack(ls, lambda p: p["ca"]["Wk"]),
        _stack(ls, lambda p: p["ca"]["Wv"].T),
        _stack(ls, lambda p: p["ca"]["bv"].reshape(1, D)),
        _stack(ls, lambda p: p["ca"]["Wo"].T),
        _stack(ls, lambda p: p["ca"]["bo"].reshape(1, D)),
        _stack(ls, lambda p: p["ln3_g"].reshape(1, D)),
        _stack(ls, lambda p: p["ln3_b"].reshape(1, D)),
        _stack(ls, lambda p: p["W1"].T),
        _stack(ls, lambda p: p["b1"].reshape(1, MLP)),
        _stack(ls, lambda p: p["W2"].T),
        _stack(ls, lambda p: p["b2"].reshape(1, D)),
    ]

    def w_spec(a):
        return pl.BlockSpec(a.shape, lambda b: (0,) * a.ndim)

    return pl.pallas_call(
        _decoder_kernel,
        grid=(B // BB,),
        in_specs=[
            pl.BlockSpec((BB, tq, D), lambda b: (b, 0, 0)),
            pl.BlockSpec((BB, TK, D), lambda b: (b, 0, 0)),
        ] + [w_spec(a) for a in ws],
        out_specs=pl.BlockSpec((BB, tq, D), lambda b: (b, 0, 0)),
        out_shape=jax.ShapeDtypeStruct((B, tq, D), jnp.float32),
        compiler_params=pltpu.CompilerParams(
            dimension_semantics=("arbitrary",),
        ),
    )(vlm_features, image_embeddings.astype(jnp.bfloat16), *ws)


# speedup vs baseline: 1.5224x; 1.5224x over previous
import jax
import jax.numpy as jnp
from jax.experimental import pallas as pl
from jax.experimental.pallas import tpu as pltpu

EMBED = 256
HEADS = 8
HD = EMBED // HEADS
MLP = 1024
LAYERS = 3
EPS = 1e-5
BB = 4
TQ = 32
QSCALE = 1.4426950408889634 / HD ** 0.5

_CT = (((1,), (1,)), ((), ()))


def _mm_t(x, w):
    return jax.lax.dot_general(x, w, _CT, preferred_element_type=jnp.float32)


def _ln(x, g, b):
    m = jnp.mean(x, axis=-1, keepdims=True)
    ms = jnp.mean(x * x, axis=-1, keepdims=True)
    v = ms - m * m
    return (x - m) * jax.lax.rsqrt(v + EPS) * g + b


def _attn(h, kvs, Wq, bq, Wk, Wv, bv, Wo, bo):
    q = (_mm_t(h, Wq) + bq) * QSCALE
    o_parts = []
    for bidx in range(BB):
        qb = q[TQ * bidx:TQ * (bidx + 1)]
        kv = kvs[bidx]
        A = jnp.concatenate(
            [qb[:, HD * i:HD * (i + 1)] @ Wk[HD * i:HD * (i + 1), :]
             for i in range(HEADS)], axis=0)
        s = _mm_t(A, kv)
        e = jnp.exp2(s)
        rs = jnp.sum(e, axis=-1, keepdims=True)
        U = jnp.dot(e, kv, preferred_element_type=jnp.float32)
        U = U * (1.0 / rs)
        o_parts.append(jnp.concatenate(
            [_mm_t(U[HD * i:HD * (i + 1), :], Wv[HD * i:HD * (i + 1), :])
             for i in range(HEADS)], axis=1))
    return _mm_t(jnp.concatenate(o_parts, axis=0) + bv, Wo) + bo


def _decoder_kernel(*refs):
    vlm_ref, img_ref = refs[0], refs[1]
    o_ref = refs[-1]
    x = vlm_ref[:].reshape(BB * TQ, EMBED)
    imgs = [img_ref[i] for i in range(BB)]
    for l in range(LAYERS):
        (ln1g, ln1b, saWq, sabq, saWk, saWv, sabv, saWo, sabo,
         ln2g, ln2b, caWq, cabq, caWk, caWv, cabv, caWo, cabo,
         ln3g, ln3b, W1, b1, W2, b2) = refs[2 + 24 * l:2 + 24 * (l + 1)]
        h = _ln(x, ln1g[:], ln1b[:])
        hs = [h[TQ * i:TQ * (i + 1)] for i in range(BB)]
        x = x + _attn(h, hs, saWq[:], sabq[:], saWk[:], saWv[:], sabv[:],
                      saWo[:], sabo[:])
        h = _ln(x, ln2g[:], ln2b[:])
        x = x + _attn(h, imgs, caWq[:], cabq[:], caWk[:], caWv[:], cabv[:],
                      caWo[:], cabo[:])
        h = _ln(x, ln3g[:], ln3b[:])
        h = jax.nn.gelu(_mm_t(h, W1[:]) + b1[:], approximate=True)
        x = x + _mm_t(h, W2[:]) + b2[:]
    o_ref[:] = x.reshape(BB, TQ, EMBED)


@jax.jit
def kernel(vlm_features, image_embeddings, params):
    B, tq, D = vlm_features.shape
    TK = image_embeddings.shape[1]

    ws = []
    for lp in params["layers"]:
        ws += [
            lp["ln1_g"].reshape(1, D), lp["ln1_b"].reshape(1, D),
            lp["sa"]["Wq"], lp["sa"]["bq"].reshape(1, D),
            lp["sa"]["Wk"], lp["sa"]["Wv"], lp["sa"]["bv"].reshape(1, D),
            lp["sa"]["Wo"], lp["sa"]["bo"].reshape(1, D),
            lp["ln2_g"].reshape(1, D), lp["ln2_b"].reshape(1, D),
            lp["ca"]["Wq"], lp["ca"]["bq"].reshape(1, D),
            lp["ca"]["Wk"], lp["ca"]["Wv"], lp["ca"]["bv"].reshape(1, D),
            lp["ca"]["Wo"], lp["ca"]["bo"].reshape(1, D),
            lp["ln3_g"].reshape(1, D), lp["ln3_b"].reshape(1, D),
            lp["W1"], lp["b1"].reshape(1, MLP),
            lp["W2"], lp["b2"].reshape(1, D),
        ]

    def w_spec(a):
        return pl.BlockSpec(a.shape, lambda b: (0,) * a.ndim)

    return pl.pallas_call(
        _decoder_kernel,
        grid=(B // BB,),
        in_specs=[
            pl.BlockSpec((BB, tq, D), lambda b: (b, 0, 0)),
            pl.BlockSpec((BB, TK, D), lambda b: (b, 0, 0)),
        ] + [w_spec(a) for a in ws],
        out_specs=pl.BlockSpec((BB, tq, D), lambda b: (b, 0, 0)),
        out_shape=jax.ShapeDtypeStruct((B, tq, D), jnp.float32),
        compiler_params=pltpu.CompilerParams(
            dimension_semantics=("arbitrary",),
        ),
    )(vlm_features, image_embeddings, *ws)


# speedup vs baseline: 1.9075x; 1.2529x over previous
import jax
import jax.numpy as jnp
from jax.experimental import pallas as pl
from jax.experimental.pallas import tpu as pltpu

EMBED = 256
HEADS = 8
HD = EMBED // HEADS
MLP = 1024
LAYERS = 3
EPS = 1e-5
BB = 4
TQ = 32
QSCALE = 1.4426950408889634 / HD ** 0.5

_CT = (((1,), (1,)), ((), ()))


def _mm_t(x, w):
    return jax.lax.dot_general(x, w, _CT, preferred_element_type=jnp.float32)


def _ln(x, g, b):
    m = jnp.mean(x, axis=-1, keepdims=True)
    ms = jnp.mean(x * x, axis=-1, keepdims=True)
    v = ms - m * m
    return (x - m) * jax.lax.rsqrt(v + EPS) * g + b


def _blockdiag_mask():
    r = jax.lax.broadcasted_iota(jnp.int32, (EMBED, EMBED), 0) // HD
    c = jax.lax.broadcasted_iota(jnp.int32, (EMBED, EMBED), 1) // HD
    return (r == c).astype(jnp.float32)


def _attn(h, kvs, Wq, bq, Wk, Wv, bv, Wo, bo):
    q = (_mm_t(h, Wq) + bq) * QSCALE
    mask = _blockdiag_mask()
    o_parts = []
    for bidx in range(BB):
        qb = q[TQ * bidx:TQ * (bidx + 1)]
        kv = kvs[bidx]
        qtile = jnp.broadcast_to(qb[None], (HEADS, TQ, EMBED)).reshape(EMBED, EMBED)
        A = (qtile * mask) @ Wk
        s = _mm_t(A, kv)
        e = jnp.exp2(s)
        rs = jnp.sum(e, axis=-1, keepdims=True)
        U = jnp.dot(e, kv, preferred_element_type=jnp.float32)
        U = U * (1.0 / rs)
        V2 = _mm_t(U, Wv) * mask
        o_parts.append(V2.reshape(HEADS, TQ, EMBED).sum(axis=0))
    return _mm_t(jnp.concatenate(o_parts, axis=0) + bv, Wo) + bo


def _decoder_kernel(*refs):
    vlm_ref, img_ref = refs[0], refs[1]
    o_ref = refs[-1]
    x = vlm_ref[:].reshape(BB * TQ, EMBED)
    imgs = [img_ref[i] for i in range(BB)]
    for l in range(LAYERS):
        (ln1g, ln1b, saWq, sabq, saWk, saWv, sabv, saWo, sabo,
         ln2g, ln2b, caWq, cabq, caWk, caWv, cabv, caWo, cabo,
         ln3g, ln3b, W1, b1, W2, b2) = refs[2 + 24 * l:2 + 24 * (l + 1)]
        h = _ln(x, ln1g[:], ln1b[:])
        hs = [h[TQ * i:TQ * (i + 1)] for i in range(BB)]
        x = x + _attn(h, hs, saWq[:], sabq[:], saWk[:], saWv[:], sabv[:],
                      saWo[:], sabo[:])
        h = _ln(x, ln2g[:], ln2b[:])
        x = x + _attn(h, imgs, caWq[:], cabq[:], caWk[:], caWv[:], cabv[:],
                      caWo[:], cabo[:])
        h = _ln(x, ln3g[:], ln3b[:])
        h = jax.nn.gelu(_mm_t(h, W1[:]) + b1[:], approximate=True)
        x = x + _mm_t(h, W2[:]) + b2[:]
    o_ref[:] = x.reshape(BB, TQ, EMBED)


@jax.jit
def kernel(vlm_features, image_embeddings, params):
    B, tq, D = vlm_features.shape
    TK = image_embeddings.shape[1]

    ws = []
    for lp in params["layers"]:
        ws += [
            lp["ln1_g"].reshape(1, D), lp["ln1_b"].reshape(1, D),
            lp["sa"]["Wq"], lp["sa"]["bq"].reshape(1, D),
            lp["sa"]["Wk"], lp["sa"]["Wv"], lp["sa"]["bv"].reshape(1, D),
            lp["sa"]["Wo"], lp["sa"]["bo"].reshape(1, D),
            lp["ln2_g"].reshape(1, D), lp["ln2_b"].reshape(1, D),
            lp["ca"]["Wq"], lp["ca"]["bq"].reshape(1, D),
            lp["ca"]["Wk"], lp["ca"]["Wv"], lp["ca"]["bv"].reshape(1, D),
            lp["ca"]["Wo"], lp["ca"]["bo"].reshape(1, D),
            lp["ln3_g"].reshape(1, D), lp["ln3_b"].reshape(1, D),
            lp["W1"], lp["b1"].reshape(1, MLP),
            lp["W2"], lp["b2"].reshape(1, D),
        ]

    def w_spec(a):
        return pl.BlockSpec(a.shape, lambda b: (0,) * a.ndim)

    return pl.pallas_call(
        _decoder_kernel,
        grid=(B // BB,),
        in_specs=[
            pl.BlockSpec((BB, tq, D), lambda b: (b, 0, 0)),
            pl.BlockSpec((BB, TK, D), lambda b: (b, 0, 0)),
        ] + [w_spec(a) for a in ws],
        out_specs=pl.BlockSpec((BB, tq, D), lambda b: (b, 0, 0)),
        out_shape=jax.ShapeDtypeStruct((B, tq, D), jnp.float32),
        compiler_params=pltpu.CompilerParams(
            dimension_semantics=("arbitrary",),
        ),
    )(vlm_features, image_embeddings, *ws)
